# baseline (speedup 1.0000x reference)
import functools

import jax
import jax.numpy as jnp
from jax import lax
from jax.experimental import pallas as pl
from jax.experimental.pallas import tpu as pltpu
from jax.experimental.pallas import tpu_sc as plsc

_IGNORE_INDEX = -100
_MAX_ITER = 3
_LANES = 16
_NUM_CORES = 2
_NUM_SUBCORES = 16


def _argmax_body(x_ref, out_ref):
    x = x_ref[0]
    pred = jnp.argmax(x, axis=-1, keepdims=True)
    out_ref[0] = pred.astype(jnp.int32)


def _sc_argmax_body(x_hbm, out_hbm, buf0, buf1, out_v, sem0, sem1,
                    *, NROWS, V, K):
    wid = lax.axis_index("s") * _NUM_CORES + lax.axis_index("c")
    nw = _NUM_CORES * _NUM_SUBCORES
    per_w = NROWS // nw
    base = wid * per_w
    nchunks = per_w // K
    bufs = (buf0, buf1)
    sems = (sem0, sem1)

    pltpu.async_copy(x_hbm.at[pl.ds(base, K)], buf0, sem0)
    pltpu.async_copy(x_hbm.at[pl.ds(base + K, K)], buf1, sem1)

    neg_inf = jnp.full((_LANES,), -jnp.inf, dtype=jnp.float32)
    zero_idx = jnp.zeros((_LANES,), dtype=jnp.int32)
    lane = lax.iota(jnp.int32, _LANES)

    def outer(jj, carry):
        for b in range(2):
            j = jj * 2 + b
            buf = bufs[b]
            pltpu.make_async_copy(x_hbm.at[pl.ds(base, K)], buf, sems[b]).wait()
            acc = zero_idx
            for r in range(K):
                def scan_row(i, c):
                    bv, bi = c
                    v = buf[r, pl.ds(i * _LANES, _LANES)]
                    idx = lane + i * _LANES
                    better = v > bv
                    return (jnp.where(better, v, bv),
                            jnp.where(better, idx, bi))
                bv, bi = lax.fori_loop(0, V // _LANES, scan_row,
                                       (neg_inf, zero_idx))
                m = jnp.max(bv)
                cand = jnp.where(bv == m, bi, jnp.int32(2 ** 30))
                acc = jnp.where(lane == r, jnp.min(cand), acc)
            out_v[pl.ds(j * K, K)] = acc

            @pl.when(j + 2 < nchunks)
            def _():
                pltpu.async_copy(
                    x_hbm.at[pl.ds(base + (j + 2) * K, K)], buf, sems[b])
        return carry

    lax.fori_loop(0, nchunks // 2, outer, jnp.int32(0))
    pltpu.sync_copy(out_v, out_hbm.at[pl.ds(base, per_w)])


def _sc_assign_body(pred_hbm, lab_hbm, valid_hbm, mask_hbm, full_hbm,
                    depth_hbm, la_hbm, full_out_hbm,
                    pred_v, lab_v, valid_v, mask_v, full_v,
                    depth_v, la_v, prop_v, pos_v, out_v, *, B, S):
    wid = lax.axis_index("s") * _NUM_CORES + lax.axis_index("c")

    @pl.when(wid < B)
    def _():
        row = wid
        pltpu.sync_copy(pred_hbm.at[row], pred_v)
        pltpu.sync_copy(lab_hbm.at[row], lab_v)
        pltpu.sync_copy(valid_hbm.at[row], valid_v)
        pltpu.sync_copy(mask_hbm.at[row], mask_v)
        pltpu.sync_copy(full_hbm.at[row], full_v)
        pltpu.sync_copy(depth_hbm, depth_v)

        d = depth_v[pl.ds(0, _LANES)]
        n_chunks = S // _LANES

        def body1(i, carry):
            sl = pl.ds(i * _LANES, _LANES)
            pred = pred_v[sl]
            lab = lab_v[sl]
            valid = valid_v[sl]
            mv = mask_v[sl]
            s_glob = lax.iota(jnp.int32, _LANES) + i * _LANES
            cont = (pred != lab) & (s_glob < S - 1) & (lab != _IGNORE_INDEX)
            la = jnp.where(cont, d + 1, d)
            la = jnp.minimum(la, _MAX_ITER)
            la = jnp.where(valid == 1, la, _IGNORE_INDEX)
            la_v[sl] = la
            prop_v[sl] = jnp.where(la == _IGNORE_INDEX, 0, la)
            cs = plsc.cumsum(mv) + carry
            pos_v[sl] = jnp.clip(cs - 1, 0, S - 1)
            return jnp.max(cs)

        lax.fori_loop(0, n_chunks, body1, jnp.int32(0))

        def body2(i, carry):
            sl = pl.ds(i * _LANES, _LANES)
            pos = pos_v[sl]
            g = plsc.load_gather(prop_v, [pos])
            mv = mask_v[sl]
            fv = full_v[sl]
            out_v[sl] = jnp.maximum(fv, jnp.where(mv != 0, g, 0))
            return carry

        lax.fori_loop(0, n_chunks, body2, jnp.int32(0))

        pltpu.sync_copy(la_v, la_hbm.at[row])
        pltpu.sync_copy(out_v, full_out_hbm.at[row])


def kernel(active_logits, active_labels_shifted, iter_depth,
           current_iter_mask, active_valid_mask, full_labels):
    B, S, V = active_logits.shape
    BS = 2048
    R = 12
    SCB = B - R
    n_sblk = S // BS

    mesh = plsc.VectorSubcoreMesh(
        core_axis_name="c", subcore_axis_name="s",
        num_cores=_NUM_CORES, num_subcores=_NUM_SUBCORES)

    pred_a = pl.pallas_call(
        _argmax_body,
        grid=(R, n_sblk),
        in_specs=[pl.BlockSpec((1, BS, V), lambda b, s: (b, s, 0))],
        out_specs=pl.BlockSpec((1, BS, 1), lambda b, s: (b * (S // BS) + s, 0, 0)),
        out_shape=jax.ShapeDtypeStruct((R * n_sblk, BS, 1), jnp.int32),
        compiler_params=pltpu.CompilerParams(
            vmem_limit_bytes=100 * 1024 * 1024),
    )(active_logits[:R])
    pred_a = pred_a.reshape(R, S)

    K = 16
    sc_rows = SCB * S
    sc_argmax = pl.kernel(
        functools.partial(_sc_argmax_body, NROWS=sc_rows, V=V, K=K),
        out_type=jax.ShapeDtypeStruct((sc_rows,), jnp.int32),
        mesh=mesh,
        scratch_types=[pltpu.VMEM((K, V), jnp.float32),
                       pltpu.VMEM((K, V), jnp.float32),
                       pltpu.VMEM((sc_rows // 32,), jnp.int32),
                       pltpu.SemaphoreType.DMA,
                       pltpu.SemaphoreType.DMA],
        compiler_params=pltpu.CompilerParams(needs_layout_passes=False),
    )
    pred_b = sc_argmax(active_logits[R:].reshape(sc_rows, V)).reshape(SCB, S)
    predicted = jnp.concatenate([pred_a, pred_b], axis=0)

    lab = active_labels_shifted.astype(jnp.int32)
    valid = active_valid_mask.astype(jnp.int32)
    maskv = current_iter_mask.astype(jnp.int32)
    full = full_labels.astype(jnp.int32)
    depth = jnp.full((_LANES,), iter_depth, dtype=jnp.int32)

    row_i32 = functools.partial(pltpu.VMEM, (S,), jnp.int32)
    sc_call = pl.kernel(
        functools.partial(_sc_assign_body, B=B, S=S),
        out_type=[jax.ShapeDtypeStruct((B, S), jnp.int32),
                  jax.ShapeDtypeStruct((B, S), jnp.int32)],
        mesh=mesh,
        scratch_types=[row_i32(), row_i32(), row_i32(), row_i32(), row_i32(),
                       pltpu.VMEM((_LANES,), jnp.int32),
                       row_i32(), row_i32(), row_i32(), row_i32()],
        compiler_params=pltpu.CompilerParams(needs_layout_passes=False),
    )
    la, full_new = sc_call(predicted, lab, valid, maskv, full, depth)
    return la, full_new


# speedup vs baseline: 1.0018x; 1.0018x over previous
import functools

import jax
import jax.numpy as jnp
from jax import lax
from jax.experimental import pallas as pl
from jax.experimental.pallas import tpu as pltpu
from jax.experimental.pallas import tpu_sc as plsc

_IGNORE_INDEX = -100
_MAX_ITER = 3
_LANES = 16
_NUM_CORES = 2
_NUM_SUBCORES = 16


def _argmax_body(x_ref, out_ref):
    x = x_ref[0]
    pred = jnp.argmax(x, axis=-1, keepdims=True)
    out_ref[0] = pred.astype(jnp.int32)


def _sc_argmax_body(x_hbm, out_hbm, buf0, buf1, out_v, sem0, sem1,
                    *, NROWS, V, K):
    wid = lax.axis_index("s") * _NUM_CORES + lax.axis_index("c")
    nw = _NUM_CORES * _NUM_SUBCORES
    per_w = NROWS // nw
    base = wid * per_w
    nchunks = per_w // K
    bufs = (buf0, buf1)
    sems = (sem0, sem1)

    pltpu.async_copy(x_hbm.at[pl.ds(base, K)], buf0, sem0)
    pltpu.async_copy(x_hbm.at[pl.ds(base + K, K)], buf1, sem1)

    neg_inf = jnp.full((_LANES,), -jnp.inf, dtype=jnp.float32)
    zero_idx = jnp.zeros((_LANES,), dtype=jnp.int32)
    lane = lax.iota(jnp.int32, _LANES)

    def outer(jj, carry):
        for b in range(2):
            j = jj * 2 + b
            buf = bufs[b]
            pltpu.make_async_copy(x_hbm.at[pl.ds(base, K)], buf, sems[b]).wait()
            acc = zero_idx
            for r in range(K):
                NACC = 4
                UNROLL = 8
                def scan_row(i, c):
                    c = list(c)
                    for u in range(UNROLL):
                        a = u % NACC
                        bv, bi = c[2 * a], c[2 * a + 1]
                        v = buf[r, pl.ds((i * UNROLL + u) * _LANES, _LANES)]
                        idx = lane + (i * UNROLL + u) * _LANES
                        better = v > bv
                        c[2 * a] = jnp.where(better, v, bv)
                        c[2 * a + 1] = jnp.where(better, idx, bi)
                    return tuple(c)
                init = (neg_inf, zero_idx) * NACC
                accs = lax.fori_loop(0, V // _LANES // UNROLL, scan_row, init)
                bv, bi = accs[0], accs[1]
                for a in range(1, NACC):
                    ov, oi = accs[2 * a], accs[2 * a + 1]
                    take = (ov > bv) | ((ov == bv) & (oi < bi))
                    bv = jnp.where(take, ov, bv)
                    bi = jnp.where(take, oi, bi)
                m = jnp.max(bv)
                cand = jnp.where(bv == m, bi, jnp.int32(2 ** 30))
                acc = jnp.where(lane == r, jnp.min(cand), acc)
            out_v[pl.ds(j * K, K)] = acc

            @pl.when(j + 2 < nchunks)
            def _():
                pltpu.async_copy(
                    x_hbm.at[pl.ds(base + (j + 2) * K, K)], buf, sems[b])
        return carry

    lax.fori_loop(0, nchunks // 2, outer, jnp.int32(0))
    pltpu.sync_copy(out_v, out_hbm.at[pl.ds(base, per_w)])


def _sc_assign_body(pred_hbm, lab_hbm, valid_hbm, mask_hbm, full_hbm,
                    depth_hbm, la_hbm, full_out_hbm,
                    pred_v, lab_v, valid_v, mask_v, full_v,
                    depth_v, la_v, prop_v, pos_v, out_v, *, B, S):
    wid = lax.axis_index("s") * _NUM_CORES + lax.axis_index("c")

    @pl.when(wid < B)
    def _():
        row = wid
        pltpu.sync_copy(pred_hbm.at[row], pred_v)
        pltpu.sync_copy(lab_hbm.at[row], lab_v)
        pltpu.sync_copy(valid_hbm.at[row], valid_v)
        pltpu.sync_copy(mask_hbm.at[row], mask_v)
        pltpu.sync_copy(full_hbm.at[row], full_v)
        pltpu.sync_copy(depth_hbm, depth_v)

        d = depth_v[pl.ds(0, _LANES)]
        n_chunks = S // _LANES

        def body1(i, carry):
            sl = pl.ds(i * _LANES, _LANES)
            pred = pred_v[sl]
            lab = lab_v[sl]
            valid = valid_v[sl]
            mv = mask_v[sl]
            s_glob = lax.iota(jnp.int32, _LANES) + i * _LANES
            cont = (pred != lab) & (s_glob < S - 1) & (lab != _IGNORE_INDEX)
            la = jnp.where(cont, d + 1, d)
            la = jnp.minimum(la, _MAX_ITER)
            la = jnp.where(valid == 1, la, _IGNORE_INDEX)
            la_v[sl] = la
            prop_v[sl] = jnp.where(la == _IGNORE_INDEX, 0, la)
            cs = plsc.cumsum(mv) + carry
            pos_v[sl] = jnp.clip(cs - 1, 0, S - 1)
            return jnp.max(cs)

        lax.fori_loop(0, n_chunks, body1, jnp.int32(0))

        def body2(i, carry):
            sl = pl.ds(i * _LANES, _LANES)
            pos = pos_v[sl]
            g = plsc.load_gather(prop_v, [pos])
            mv = mask_v[sl]
            fv = full_v[sl]
            out_v[sl] = jnp.maximum(fv, jnp.where(mv != 0, g, 0))
            return carry

        lax.fori_loop(0, n_chunks, body2, jnp.int32(0))

        pltpu.sync_copy(la_v, la_hbm.at[row])
        pltpu.sync_copy(out_v, full_out_hbm.at[row])


def kernel(active_logits, active_labels_shifted, iter_depth,
           current_iter_mask, active_valid_mask, full_labels):
    B, S, V = active_logits.shape
    BS = 2048
    R = 12
    SCB = B - R
    n_sblk = S // BS

    mesh = plsc.VectorSubcoreMesh(
        core_axis_name="c", subcore_axis_name="s",
        num_cores=_NUM_CORES, num_subcores=_NUM_SUBCORES)

    pred_a = pl.pallas_call(
        _argmax_body,
        grid=(R, n_sblk),
        in_specs=[pl.BlockSpec((1, BS, V), lambda b, s: (b, s, 0))],
        out_specs=pl.BlockSpec((1, BS, 1), lambda b, s: (b * (S // BS) + s, 0, 0)),
        out_shape=jax.ShapeDtypeStruct((R * n_sblk, BS, 1), jnp.int32),
        compiler_params=pltpu.CompilerParams(
            vmem_limit_bytes=100 * 1024 * 1024),
    )(active_logits[:R])
    pred_a = pred_a.reshape(R, S)

    K = 16
    sc_rows = SCB * S
    sc_argmax = pl.kernel(
        functools.partial(_sc_argmax_body, NROWS=sc_rows, V=V, K=K),
        out_type=jax.ShapeDtypeStruct((sc_rows,), jnp.int32),
        mesh=mesh,
        scratch_types=[pltpu.VMEM((K, V), jnp.float32),
                       pltpu.VMEM((K, V), jnp.float32),
                       pltpu.VMEM((sc_rows // 32,), jnp.int32),
                       pltpu.SemaphoreType.DMA,
                       pltpu.SemaphoreType.DMA],
        compiler_params=pltpu.CompilerParams(needs_layout_passes=False),
    )
    pred_b = sc_argmax(active_logits[R:].reshape(sc_rows, V)).reshape(SCB, S)
    predicted = jnp.concatenate([pred_a, pred_b], axis=0)

    lab = active_labels_shifted.astype(jnp.int32)
    valid = active_valid_mask.astype(jnp.int32)
    maskv = current_iter_mask.astype(jnp.int32)
    full = full_labels.astype(jnp.int32)
    depth = jnp.full((_LANES,), iter_depth, dtype=jnp.int32)

    row_i32 = functools.partial(pltpu.VMEM, (S,), jnp.int32)
    sc_call = pl.kernel(
        functools.partial(_sc_assign_body, B=B, S=S),
        out_type=[jax.ShapeDtypeStruct((B, S), jnp.int32),
                  jax.ShapeDtypeStruct((B, S), jnp.int32)],
        mesh=mesh,
        scratch_types=[row_i32(), row_i32(), row_i32(), row_i32(), row_i32(),
                       pltpu.VMEM((_LANES,), jnp.int32),
                       row_i32(), row_i32(), row_i32(), row_i32()],
        compiler_params=pltpu.CompilerParams(needs_layout_passes=False),
    )
    la, full_new = sc_call(predicted, lab, valid, maskv, full, depth)
    return la, full_new


# speedup vs baseline: 2.3700x; 2.3657x over previous
import functools

import jax
import jax.numpy as jnp
from jax import lax
from jax.experimental import pallas as pl
from jax.experimental.pallas import tpu as pltpu
from jax.experimental.pallas import tpu_sc as plsc

_IGNORE_INDEX = -100
_MAX_ITER = 3
_LANES = 16
_NUM_CORES = 2
_NUM_SUBCORES = 16


def _argmax_body(x_ref, out_ref):
    x = x_ref[0]
    pred = jnp.argmax(x, axis=-1, keepdims=True)
    out_ref[0] = pred.astype(jnp.int32)


def _sc_assign_body(pred_hbm, lab_hbm, valid_hbm, mask_hbm, full_hbm,
                    depth_hbm, la_hbm, full_out_hbm,
                    pred_v, lab_v, valid_v, mask_v, full_v,
                    depth_v, la_v, prop_v, pos_v, out_v, *, B, S):
    cid = lax.axis_index("c")
    sid = lax.axis_index("s")

    @pl.when((cid == 0) & (sid < B))
    def _():
        row = sid
        pltpu.sync_copy(pred_hbm.at[row], pred_v)
        pltpu.sync_copy(lab_hbm.at[row], lab_v)
        pltpu.sync_copy(valid_hbm.at[row], valid_v)
        pltpu.sync_copy(mask_hbm.at[row], mask_v)
        pltpu.sync_copy(full_hbm.at[row], full_v)
        pltpu.sync_copy(depth_hbm, depth_v)

        d = depth_v[pl.ds(0, _LANES)]
        n_chunks = S // _LANES

        def body1(i, carry):
            sl = pl.ds(i * _LANES, _LANES)
            pred = pred_v[sl]
            lab = lab_v[sl]
            valid = valid_v[sl]
            mv = mask_v[sl]
            s_glob = lax.iota(jnp.int32, _LANES) + i * _LANES
            cont = (pred != lab) & (s_glob < S - 1) & (lab != _IGNORE_INDEX)
            la = jnp.where(cont, d + 1, d)
            la = jnp.minimum(la, _MAX_ITER)
            la = jnp.where(valid == 1, la, _IGNORE_INDEX)
            la_v[sl] = la
            prop_v[sl] = jnp.where(la == _IGNORE_INDEX, 0, la)
            cs = plsc.cumsum(mv) + carry
            pos_v[sl] = jnp.clip(cs - 1, 0, S - 1)
            return jnp.max(cs)

        lax.fori_loop(0, n_chunks, body1, jnp.int32(0))

        def body2(i, carry):
            sl = pl.ds(i * _LANES, _LANES)
            pos = pos_v[sl]
            g = plsc.load_gather(prop_v, [pos])
            mv = mask_v[sl]
            fv = full_v[sl]
            out_v[sl] = jnp.maximum(fv, jnp.where(mv != 0, g, 0))
            return carry

        lax.fori_loop(0, n_chunks, body2, jnp.int32(0))

        pltpu.sync_copy(la_v, la_hbm.at[row])
        pltpu.sync_copy(out_v, full_out_hbm.at[row])


def kernel(active_logits, active_labels_shifted, iter_depth,
           current_iter_mask, active_valid_mask, full_labels):
    B, S, V = active_logits.shape
    BS = 2048
    n_sblk = S // BS

    predicted = pl.pallas_call(
        _argmax_body,
        grid=(B, n_sblk),
        in_specs=[pl.BlockSpec((1, BS, V), lambda b, s: (b, s, 0))],
        out_specs=pl.BlockSpec((1, BS, 1), lambda b, s: (b * (S // BS) + s, 0, 0)),
        out_shape=jax.ShapeDtypeStruct((B * n_sblk, BS, 1), jnp.int32),
        compiler_params=pltpu.CompilerParams(
            vmem_limit_bytes=100 * 1024 * 1024),
    )(active_logits)
    predicted = predicted.reshape(B, S)

    lab = active_labels_shifted.astype(jnp.int32)
    valid = active_valid_mask.astype(jnp.int32)
    maskv = current_iter_mask.astype(jnp.int32)
    full = full_labels.astype(jnp.int32)
    depth = jnp.full((_LANES,), iter_depth, dtype=jnp.int32)

    mesh = plsc.VectorSubcoreMesh(
        core_axis_name="c", subcore_axis_name="s",
        num_cores=_NUM_CORES, num_subcores=_NUM_SUBCORES)
    row_i32 = functools.partial(pltpu.VMEM, (S,), jnp.int32)
    sc_call = pl.kernel(
        functools.partial(_sc_assign_body, B=B, S=S),
        out_type=[jax.ShapeDtypeStruct((B, S), jnp.int32),
                  jax.ShapeDtypeStruct((B, S), jnp.int32)],
        mesh=mesh,
        scratch_types=[row_i32(), row_i32(), row_i32(), row_i32(), row_i32(),
                       pltpu.VMEM((_LANES,), jnp.int32),
                       row_i32(), row_i32(), row_i32(), row_i32()],
        compiler_params=pltpu.CompilerParams(needs_layout_passes=False),
    )
    la, full_new = sc_call(predicted, lab, valid, maskv, full, depth)
    return la, full_new
